# Initial kernel scaffold; baseline (speedup 1.0000x reference)
import jax
import jax.numpy as jnp
from jax.experimental import pallas as pl


def kernel(demo_input_grids, demo_input_masks, demo_output_grids, demo_output_masks, demo_mask, query_input_grid, query_input_mask):
    raise NotImplementedError("write your pallas kernel here")


# speedup vs baseline: 9.8288x; 9.8288x over previous
import numpy as np
import jax
import jax.numpy as jnp
from jax.experimental import pallas as pl

G = 30
C = 10
BIG = 1000000000.0

_LANES = np.arange(G * C)
_R_NP = (np.equal.outer(np.arange(G), _LANES // C)).astype(np.float32)
_COL_NP = np.broadcast_to((_LANES % C).astype(np.float32), (8, G * C)).copy()


def _body(g2_ref, m2_ref, m3_ref, r_ref, col_ref, hl_ref, wl_ref, cell_ref):
    g = g2_ref[...]
    m = m2_ref[...]
    safe = jnp.where(m, g, 0).astype(jnp.bfloat16)
    rep = jax.lax.dot_general(
        safe, r_ref[...], (((1,), (0,)), ((), ())),
        preferred_element_type=jnp.float32)
    eq = rep == col_ref[0:1, :]
    cell_ref[...] = jnp.where(eq, BIG, -BIG)

    m3 = m3_ref[...]
    bb = m3.shape[0]
    row_any = jnp.any(m3, axis=2)
    col_any = jnp.any(m3, axis=1)
    h = jnp.sum(row_any.astype(jnp.int32), axis=1) - 1
    w = jnp.sum(col_any.astype(jnp.int32), axis=1) - 1
    h = jnp.where(h < 0, h + G, h)
    w = jnp.where(w < 0, w + G, w)
    iot = jax.lax.broadcasted_iota(jnp.int32, (bb, G), 1)
    hl_ref[...] = jnp.where(iot == h[:, None], BIG, -BIG)
    wl_ref[...] = jnp.where(iot == w[:, None], BIG, -BIG)


def _build(B, Bb, interpret=False):
    grid = (B // Bb,)
    return pl.pallas_call(
        _body,
        grid=grid,
        in_specs=[
            pl.BlockSpec((Bb * G, G), lambda i: (i, 0)),
            pl.BlockSpec((Bb * G, G), lambda i: (i, 0)),
            pl.BlockSpec((Bb, G, G), lambda i: (i, 0, 0)),
            pl.BlockSpec((G, G * C), lambda i: (0, 0)),
            pl.BlockSpec((8, G * C), lambda i: (0, 0)),
        ],
        out_specs=[
            pl.BlockSpec((Bb, G), lambda i: (i, 0)),
            pl.BlockSpec((Bb, G), lambda i: (i, 0)),
            pl.BlockSpec((Bb * G, G * C), lambda i: (i, 0)),
        ],
        out_shape=[
            jax.ShapeDtypeStruct((B, G), jnp.float32),
            jax.ShapeDtypeStruct((B, G), jnp.float32),
            jax.ShapeDtypeStruct((B * G, G * C), jnp.float32),
        ],
        interpret=interpret,
    )


def kernel(demo_input_grids, demo_input_masks, demo_output_grids,
           demo_output_masks, demo_mask, query_input_grid, query_input_mask):
    del demo_input_grids, demo_input_masks, demo_output_grids
    del demo_output_masks, demo_mask
    B = query_input_grid.shape[0]
    Bb = 128
    g2 = query_input_grid.reshape(B * G, G)
    m2 = query_input_mask.reshape(B * G, G)
    r_mat = jnp.asarray(_R_NP, dtype=jnp.bfloat16)
    col = jnp.asarray(_COL_NP)
    hl, wl, cell2 = _build(B, Bb)(g2, m2, query_input_mask, r_mat, col)
    return (hl, wl, cell2.reshape(B, G, G, C))
